# Initial kernel scaffold; baseline (speedup 1.0000x reference)
import jax
import jax.numpy as jnp
from jax.experimental import pallas as pl


def kernel(x, edge_index, W, b):
    raise NotImplementedError("write your pallas kernel here")


# speedup vs baseline: 11.9548x; 11.9548x over previous
import functools

import jax
import jax.numpy as jnp
from jax import lax
from jax.experimental import pallas as pl
from jax.experimental.pallas import tpu as pltpu
from jax.experimental.pallas import tpu_sc as plsc

N = 10000
E = 320000
D = 128
NC = 2
NS = 16
NW = NC * NS
EPW = E // NW
CHUNK = 80
NCHUNK = EPW // CHUNK
NPAD = 10240
RPT = NPAD // NS
DEGW = 128

_sc_mesh = plsc.VectorSubcoreMesh(core_axis_name="c", subcore_axis_name="s")


@functools.partial(
    pl.kernel,
    out_type=jax.ShapeDtypeStruct((NC, NPAD, DEGW), jnp.float32),
    mesh=_sc_mesh,
    scratch_types=[
        pltpu.VMEM((CHUNK,), jnp.int32),
        pltpu.VMEM((CHUNK, DEGW), jnp.float32),
        pltpu.VMEM_SHARED((NPAD, DEGW), jnp.float32),
    ],
)
def _deg_kernel(col_hbm, ones_hbm, zeros_hbm, out_hbm, idx_c, ones_v, tab):
    c = lax.axis_index("c")
    s = lax.axis_index("s")
    pltpu.sync_copy(zeros_hbm, tab.at[pl.ds(s * RPT, RPT)])
    pltpu.sync_copy(ones_hbm, ones_v)
    plsc.subcore_barrier()
    base = (c * NS + s) * EPW

    def body(i, carry):
        off = pl.multiple_of(base + i * CHUNK, 8)
        pltpu.sync_copy(col_hbm.at[pl.ds(off, CHUNK)], idx_c)
        pltpu.sync_copy(ones_v, tab.at[idx_c], add=True)
        return carry

    lax.fori_loop(0, NCHUNK, body, 0)
    plsc.subcore_barrier()
    pltpu.sync_copy(tab.at[pl.ds(s * RPT, RPT)],
                    out_hbm.at[c, pl.ds(s * RPT, RPT)])


@functools.partial(
    pl.kernel,
    out_type=jax.ShapeDtypeStruct((NC, NPAD, D), jnp.float32),
    mesh=_sc_mesh,
    scratch_types=[
        pltpu.VMEM((CHUNK,), jnp.int32),
        pltpu.VMEM((CHUNK,), jnp.int32),
        pltpu.VMEM((CHUNK, D), jnp.float32),
        pltpu.VMEM_SHARED((NPAD, D), jnp.float32),
        pltpu.SemaphoreType.DMA,
    ],
)
def _hop_kernel(u_hbm, row_hbm, col_hbm, zeros_hbm, out_hbm,
                idx_r, idx_c, rows_v, acc, sem):
    c = lax.axis_index("c")
    s = lax.axis_index("s")
    pltpu.sync_copy(zeros_hbm, acc.at[pl.ds(s * RPT, RPT)])
    plsc.subcore_barrier()
    base = (c * NS + s) * EPW

    def body(i, carry):
        off = pl.multiple_of(base + i * CHUNK, 8)
        pltpu.sync_copy(row_hbm.at[pl.ds(off, CHUNK)], idx_r)
        pltpu.sync_copy(col_hbm.at[pl.ds(off, CHUNK)], idx_c)
        pltpu.async_copy(u_hbm.at[idx_r], rows_v, sem).wait()
        pltpu.sync_copy(rows_v, acc.at[idx_c], add=True)
        return carry

    lax.fori_loop(0, NCHUNK, body, 0)
    plsc.subcore_barrier()
    pltpu.sync_copy(acc.at[pl.ds(s * RPT, RPT)],
                    out_hbm.at[c, pl.ds(s * RPT, RPT)])


_R = 2000
_GRID = N // _R


def _deg_of(d0, d1):
    return d0[:, :1] + d1[:, :1] + 1.0


def _mm_body(x_ref, wt_ref, d0_ref, d1_ref, y_ref):
    dinv = lax.rsqrt(_deg_of(d0_ref[...], d1_ref[...]))
    y_ref[...] = dinv * jnp.dot(x_ref[...], wt_ref[...],
                                preferred_element_type=jnp.float32)


def _comb_body(p0_ref, p1_ref, u_ref, d0_ref, d1_ref, o_ref):
    deg = _deg_of(d0_ref[...], d1_ref[...])
    o_ref[...] = (p0_ref[...] + p1_ref[...] + u_ref[...]) / deg


def _final_body(p0_ref, p1_ref, u_ref, d0_ref, d1_ref, b_ref, o_ref):
    dinv = lax.rsqrt(_deg_of(d0_ref[...], d1_ref[...]))
    o_ref[...] = (p0_ref[...] + p1_ref[...] + u_ref[...]) * dinv + b_ref[...]


_row_spec = pl.BlockSpec((_R, D), lambda i: (i, 0))
_deg_spec = pl.BlockSpec((_R, DEGW), lambda i: (i, 0))
_w_spec = pl.BlockSpec((D, D), lambda i: (0, 0))
_b_spec = pl.BlockSpec((1, D), lambda i: (0, 0))
_out_row = jax.ShapeDtypeStruct((N, D), jnp.float32)

_mm_call = pl.pallas_call(
    _mm_body, grid=(_GRID,),
    in_specs=[_row_spec, _w_spec, _deg_spec, _deg_spec],
    out_specs=_row_spec, out_shape=_out_row)

_comb_call = pl.pallas_call(
    _comb_body, grid=(_GRID,),
    in_specs=[_row_spec, _row_spec, _row_spec, _deg_spec, _deg_spec],
    out_specs=_row_spec, out_shape=_out_row)

_final_call = pl.pallas_call(
    _final_body, grid=(_GRID,),
    in_specs=[_row_spec, _row_spec, _row_spec, _deg_spec, _deg_spec, _b_spec],
    out_specs=_row_spec, out_shape=_out_row)


def kernel(x, edge_index, W, b):
    row = edge_index[0]
    col = edge_index[1]
    wt = W.T
    ones_deg = jnp.ones((CHUNK, DEGW), jnp.float32)
    zeros_deg = jnp.zeros((RPT, DEGW), jnp.float32)
    zeros_row = jnp.zeros((RPT, D), jnp.float32)
    b2 = b.reshape(1, D)

    degp = _deg_kernel(col, ones_deg, zeros_deg)
    d0, d1 = degp[0], degp[1]

    u0 = _mm_call(x, wt, d0, d1)
    p = _hop_kernel(u0, row, col, zeros_row)
    u1 = _comb_call(p[0], p[1], u0, d0, d1)
    p = _hop_kernel(u1, row, col, zeros_row)
    out = _final_call(p[0], p[1], u1, d0, d1, b2)
    return (out, out)
